# Initial kernel scaffold; baseline (speedup 1.0000x reference)
import jax
import jax.numpy as jnp
from jax.experimental import pallas as pl


def kernel(x, edge_index, edge_weight, W, b, W2, b2):
    raise NotImplementedError("write your pallas kernel here")


# speedup vs baseline: 24.5103x; 24.5103x over previous
import functools

import jax
import jax.numpy as jnp
from jax import lax
from jax.experimental import pallas as pl
from jax.experimental.pallas import tpu as pltpu
from jax.experimental.pallas import tpu_sc as plsc

NNODES = 10000
NEDGES = 320000
DM = 128
NC, NS = 2, 16
NW = NC * NS
ECHUNK = 128
ROWS_PER_W = 80
PADROWS = NW * ROWS_PER_W
PADE = PADROWS * ECHUNK - NEDGES
NACC = 10240
TROWS = NACC // NS
TAILROWS = NNODES - 15 * TROWS

_MESH = plsc.VectorSubcoreMesh(core_axis_name="c", subcore_axis_name="s")


@functools.partial(
    pl.kernel,
    out_type=jax.ShapeDtypeStruct((NC * NNODES,), jnp.float32),
    mesh=_MESH,
    scratch_types=[
        pltpu.VMEM((ROWS_PER_W, ECHUNK), jnp.int32),
        pltpu.VMEM((ROWS_PER_W, ECHUNK), jnp.float32),
        pltpu.VMEM((NNODES,), jnp.float32),
        pltpu.VMEM_SHARED((NNODES,), jnp.float32),
    ],
)
def _deg_kernel(dst_hbm, ew_hbm, degp_hbm, idx_v, ew_v, zbuf_v, deg_sh):
    c = lax.axis_index("c")
    s = lax.axis_index("s")

    @pl.when(s == 0)
    def _():
        z16 = jnp.zeros((16,), jnp.float32)

        def zbody(i, carry):
            zbuf_v[pl.ds(i * 16, 16)] = z16
            return carry

        lax.fori_loop(0, NNODES // 16, zbody, 0)
        pltpu.sync_copy(zbuf_v, deg_sh)

    plsc.subcore_barrier()

    base = (c * NS + s) * ROWS_PER_W
    pltpu.sync_copy(dst_hbm.at[pl.ds(base, ROWS_PER_W)], idx_v)
    pltpu.sync_copy(ew_hbm.at[pl.ds(base, ROWS_PER_W)], ew_v)

    def deg_row(j, carry):
        pltpu.sync_copy(ew_v.at[j], deg_sh.at[idx_v.at[j]], add=True)
        return carry

    lax.fori_loop(0, ROWS_PER_W, deg_row, 0)

    plsc.subcore_barrier()

    @pl.when(s == 0)
    def _():
        pltpu.sync_copy(deg_sh, zbuf_v)
        pltpu.sync_copy(zbuf_v, degp_hbm.at[pl.ds(c * NNODES, NNODES)])


@functools.partial(
    pl.kernel,
    out_type=jax.ShapeDtypeStruct((NC, NNODES, DM), jnp.float32),
    mesh=_MESH,
    scratch_types=[
        pltpu.VMEM((ROWS_PER_W, ECHUNK), jnp.int32),
        pltpu.VMEM((ROWS_PER_W, ECHUNK), jnp.int32),
        pltpu.VMEM((ROWS_PER_W, ECHUNK), jnp.float32),
        pltpu.VMEM((ECHUNK, DM), jnp.float32),
        pltpu.SemaphoreType.DMA,
        pltpu.VMEM_SHARED((NACC, DM), jnp.float32),
    ],
)
def _agg_kernel(src_hbm, dst_hbm, ew_hbm, z_hbm, aggp_hbm,
                sidx_v, didx_v, ew_v, rows_v, gsem, acc_sh):
    c = lax.axis_index("c")
    s = lax.axis_index("s")

    z16 = jnp.zeros((16,), jnp.float32)

    def zbody(i, carry):
        rows_v[i // 8, pl.ds((i % 8) * 16, 16)] = z16
        return carry

    lax.fori_loop(0, ECHUNK * (DM // 16), zbody, 0)
    for k in range(TROWS // ECHUNK):
        pltpu.sync_copy(rows_v, acc_sh.at[pl.ds(s * TROWS + k * ECHUNK, ECHUNK)])
    plsc.subcore_barrier()

    base = (c * NS + s) * ROWS_PER_W
    pltpu.sync_copy(src_hbm.at[pl.ds(base, ROWS_PER_W)], sidx_v)
    pltpu.sync_copy(dst_hbm.at[pl.ds(base, ROWS_PER_W)], didx_v)
    pltpu.sync_copy(ew_hbm.at[pl.ds(base, ROWS_PER_W)], ew_v)

    def edge_row(j, carry):
        pltpu.async_copy(z_hbm.at[sidx_v.at[j]], rows_v, gsem).wait()

        def scale_block(eb, carry2):
            wv = ew_v[j, pl.ds(eb * 16, 16)]
            for l in range(16):
                wgt = wv[l]
                e = eb * 16 + l
                for g in range(DM // 16):
                    sl = pl.ds(g * 16, 16)
                    rows_v[e, sl] = rows_v[e, sl] * wgt
            return carry2

        lax.fori_loop(0, ECHUNK // 16, scale_block, 0)
        pltpu.sync_copy(rows_v, acc_sh.at[didx_v.at[j]], add=True)
        return carry

    lax.fori_loop(0, ROWS_PER_W, edge_row, 0)

    plsc.subcore_barrier()

    @pl.when(s < NS - 1)
    def _():
        pltpu.sync_copy(acc_sh.at[pl.ds(s * TROWS, TROWS)],
                        aggp_hbm.at[c, pl.ds(s * TROWS, TROWS)])

    @pl.when(s == NS - 1)
    def _():
        pltpu.sync_copy(acc_sh.at[pl.ds((NS - 1) * TROWS, TAILROWS)],
                        aggp_hbm.at[c, pl.ds((NS - 1) * TROWS, TAILROWS)])


def _prep_body(x_ref, w_ref, degp_ref, z_ref):
    deg = 1.0 + degp_ref[0] + degp_ref[1]
    dinv = jnp.where(deg > 0, lax.rsqrt(deg), 0.0)
    xw = jnp.dot(x_ref[...], w_ref[...], preferred_element_type=jnp.float32)
    z_ref[...] = xw * dinv


_prep = pl.pallas_call(
    _prep_body,
    out_shape=jax.ShapeDtypeStruct((NNODES, DM), jnp.float32),
)


def _final_body(aggp_ref, z_ref, degp_ref, b_ref, w2_ref, b2_ref, y_ref):
    deg = 1.0 + degp_ref[0] + degp_ref[1]
    dinv = jnp.where(deg > 0, lax.rsqrt(deg), 0.0)
    h = (aggp_ref[0] + aggp_ref[1] + z_ref[...]) * dinv + b_ref[...]
    h = jnp.maximum(h, 0.0)
    y = jnp.dot(h, w2_ref[...], preferred_element_type=jnp.float32) + b2_ref[...]
    y_ref[...] = jax.nn.sigmoid(y)


_final = pl.pallas_call(
    _final_body,
    out_shape=jax.ShapeDtypeStruct((NNODES, 2), jnp.float32),
)


def kernel(x, edge_index, edge_weight, W, b, W2, b2):
    padi = (jnp.arange(PADE, dtype=jnp.int32) * 13) % NNODES
    src = jnp.concatenate([edge_index[0].astype(jnp.int32), padi]).reshape(PADROWS, ECHUNK)
    dst = jnp.concatenate([edge_index[1].astype(jnp.int32), padi]).reshape(PADROWS, ECHUNK)
    ew = jnp.concatenate([edge_weight, jnp.zeros((PADE,), jnp.float32)]).reshape(PADROWS, ECHUNK)
    degp = _deg_kernel(dst, ew)
    degp3 = degp.reshape(NC, NNODES, 1)
    z = _prep(x, W, degp3)
    aggp = _agg_kernel(src, dst, ew, z)
    return _final(aggp, z, degp3, b.reshape(1, DM), W2, b2.reshape(1, 2))
